# Initial kernel scaffold; baseline (speedup 1.0000x reference)
import jax
import jax.numpy as jnp
from jax.experimental import pallas as pl


def kernel(x, coords, conv1_w, bn1_g, bn1_b, conv2_w, bn2_g, bn2_b, convE1_w, bnE1_g, bnE1_b, convE2_w, bnE2_g, bnE2_b):
    raise NotImplementedError("write your pallas kernel here")


# speedup vs baseline: 11.8530x; 11.8530x over previous
import functools

import jax
import jax.numpy as jnp
from jax.experimental import pallas as pl

S = 512
K = 22
KPAD = 32
BN_EPS = 1e-5


def _fps_kernel(ct_ref, fidx_ref, cent_ref, *, s_count, n):
    xr = ct_ref[:, 0, :]
    yr = ct_ref[:, 1, :]
    zr = ct_ref[:, 2, :]
    b = xr.shape[0]
    iota_n = jax.lax.broadcasted_iota(jnp.int32, (b, n), 1)
    iota_nf = iota_n.astype(jnp.float32)
    iota_s = jax.lax.broadcasted_iota(jnp.int32, (b, s_count), 1)

    ones_n = xr * 0.0 + 1.0
    zeros_s = xr[:, 0:s_count] * 0.0

    dist0 = xr * 0.0 + 1e10
    far0 = jnp.zeros((b, 1), dtype=jnp.int32)
    fidx0 = zeros_s
    rec0 = zeros_s

    def body(s, carry):
        dist, far, fidx, cxr, cyr, czr = carry
        self_f = (iota_n == far).astype(jnp.float32) * ones_n
        cx = jnp.sum(self_f * xr, axis=1, keepdims=True)
        cy = jnp.sum(self_f * yr, axis=1, keepdims=True)
        cz = jnp.sum(self_f * zr, axis=1, keepdims=True)
        recsf = (iota_s == s).astype(jnp.float32) + zeros_s
        fidx = fidx + recsf * far.astype(jnp.float32)
        cxr = cxr + recsf * cx
        cyr = cyr + recsf * cy
        czr = czr + recsf * cz
        dx = xr - cx
        dy = yr - cy
        dz = zr - cz
        d = dx * dx + dy * dy + dz * dz
        dist = jnp.minimum(dist, d)
        m = jnp.max(dist, axis=1, keepdims=True)
        far = jnp.min(
            jnp.where(dist == m, iota_nf, float(n)), axis=1, keepdims=True
        ).astype(jnp.int32)
        return dist, far, fidx, cxr, cyr, czr

    _, _, fidx, cxr, cyr, czr = jax.lax.fori_loop(
        0, s_count, body, (dist0, far0, fidx0, rec0, rec0, rec0)
    )
    fidx_ref[...] = fidx.astype(jnp.int32)
    cent_ref[:, 0, :] = cxr
    cent_ref[:, 1, :] = cyr
    cent_ref[:, 2, :] = czr


def _knn_kernel(ct_ref, cent_ref, nidx_ref, *, s_count, k, kpad, n):
    x3 = ct_ref[0]
    c3 = cent_ref[0]
    cT = jnp.transpose(c3)
    xsq = jnp.sum(x3 * x3, axis=0, keepdims=True)
    csq = jnp.sum(cT * cT, axis=1, keepdims=True)
    dotm = jnp.dot(cT, x3, preferred_element_type=jnp.float32)
    d = csq + xsq - 2.0 * dotm
    iota_n = jax.lax.broadcasted_iota(jnp.int32, (s_count, n), 1)
    iota_nf = iota_n.astype(jnp.float32)
    cols = []
    for _ in range(k):
        m = jnp.min(d, axis=1, keepdims=True)
        idx = jnp.min(
            jnp.where(d == m, iota_nf, float(n)), axis=1, keepdims=True
        ).astype(jnp.int32)
        cols.append(idx)
        d = jnp.where(iota_n == idx, 1e30, d)
    cols.append(jnp.zeros((s_count, kpad - k), dtype=jnp.int32))
    nidx_ref[0] = jnp.concatenate(cols, axis=1)


def _c1_kernel(f_ref, nidx_ref, w1t_ref, we1t_ref, z_ref, cnt_ref, acc_ref,
               *, s_count, k, n):
    b = pl.program_id(0)
    f = f_ref[0]
    nidx = nidx_ref[0]
    z1 = jnp.dot(f[:, 0:64], w1t_ref[...], preferred_element_type=jnp.float32)
    z1e = jnp.dot(f[:, 64:128], we1t_ref[...],
                  preferred_element_type=jnp.float32)
    z = jnp.concatenate([z1, z1e], axis=1)
    z_ref[0] = z
    iota_n = jax.lax.broadcasted_iota(jnp.int32, (s_count, n), 1)
    counts = jnp.zeros((1, n), dtype=jnp.float32)
    for kk in range(k):
        maskk = (nidx[:, kk:kk + 1] == iota_n).astype(jnp.float32)
        counts = counts + jnp.sum(maskk, axis=0, keepdims=True)
    cnt_ref[0] = counts
    s1 = jnp.dot(counts, z, preferred_element_type=jnp.float32)
    s2 = jnp.dot(counts, z * z, preferred_element_type=jnp.float32)
    pad = jnp.zeros((6, 128), dtype=jnp.float32)
    upd = jnp.concatenate([s1, s2, pad], axis=0)

    @pl.when(b == 0)
    def _():
        acc_ref[0] = jnp.zeros_like(acc_ref[0])

    acc_ref[0] += upd


def _bn_scale_shift(acc, g_row, b_row, total):
    mean = acc[0:1, :] / total
    var = acc[1:2, :] / total - mean * mean
    scale = jax.lax.rsqrt(var + BN_EPS) * g_row
    shift = b_row - mean * scale
    return scale, shift


def _c2_kernel(z_ref, cnt_ref, acc1_ref, g1_ref, b1_ref, w2t_ref, we2t_ref,
               z2_ref, acc2_ref, *, total):
    b = pl.program_id(0)
    scale, shift = _bn_scale_shift(acc1_ref[0], g1_ref[0], b1_ref[0], total)
    a1 = jnp.maximum(z_ref[0] * scale + shift, 0.0)
    z2 = jnp.dot(a1[:, 0:64], w2t_ref[...], preferred_element_type=jnp.float32)
    z2e = jnp.dot(a1[:, 64:128], we2t_ref[...],
                  preferred_element_type=jnp.float32)
    z2c = jnp.concatenate([z2, z2e], axis=1)
    z2_ref[0] = z2c
    counts = cnt_ref[0]
    s1 = jnp.dot(counts, z2c, preferred_element_type=jnp.float32)
    s2 = jnp.dot(counts, z2c * z2c, preferred_element_type=jnp.float32)
    pad = jnp.zeros((6, 128), dtype=jnp.float32)
    upd = jnp.concatenate([s1, s2, pad], axis=0)

    @pl.when(b == 0)
    def _():
        acc2_ref[0] = jnp.zeros_like(acc2_ref[0])

    acc2_ref[0] += upd


def _c3_kernel(z2_ref, nidx_ref, acc2_ref, g2_ref, b2_ref, fk_ref,
               *, s_count, k, n, total):
    scale, shift = _bn_scale_shift(acc2_ref[0], g2_ref[0], b2_ref[0], total)
    a2 = jnp.maximum(z2_ref[0] * scale + shift, 0.0)
    nidx = nidx_ref[0]
    iota_n = jax.lax.broadcasted_iota(jnp.int32, (s_count, n), 1)
    fk = jnp.full((s_count, 128), -1e30, dtype=jnp.float32)
    for kk in range(k):
        maskk = (nidx[:, kk:kk + 1] == iota_n).astype(jnp.float32)
        gk = jnp.dot(maskk, a2, preferred_element_type=jnp.float32)
        fk = jnp.maximum(fk, gk)
    fk_ref[0] = fk


def _run(x, coords, conv1_w, bn1_g, bn1_b, conv2_w, bn2_g, bn2_b,
         convE1_w, bnE1_g, bnE1_b, convE2_w, bnE2_g, bnE2_b):
    B, C, N = x.shape
    total = float(B * S * K)

    coordsT = jnp.transpose(coords, (0, 2, 1))
    fk_feat = jnp.transpose(x[:, 0:128, :], (0, 2, 1))
    w1t = jnp.transpose(conv1_w)
    we1t = jnp.transpose(convE1_w)
    w2t = jnp.transpose(conv2_w)
    we2t = jnp.transpose(convE2_w)
    g1 = jnp.concatenate([bn1_g, bnE1_g]).reshape(1, 1, 128)
    b1 = jnp.concatenate([bn1_b, bnE1_b]).reshape(1, 1, 128)
    g2 = jnp.concatenate([bn2_g, bnE2_g]).reshape(1, 1, 128)
    b2 = jnp.concatenate([bn2_b, bnE2_b]).reshape(1, 1, 128)

    fidx, centT = pl.pallas_call(
        functools.partial(_fps_kernel, s_count=S, n=N),
        out_shape=(
            jax.ShapeDtypeStruct((B, S), jnp.int32),
            jax.ShapeDtypeStruct((B, 3, S), jnp.float32),
        ),
    )(coordsT)
    del fidx

    nidx = pl.pallas_call(
        functools.partial(_knn_kernel, s_count=S, k=K, kpad=KPAD, n=N),
        grid=(B,),
        in_specs=[
            pl.BlockSpec((1, 3, N), lambda b: (b, 0, 0)),
            pl.BlockSpec((1, 3, S), lambda b: (b, 0, 0)),
        ],
        out_specs=pl.BlockSpec((1, S, KPAD), lambda b: (b, 0, 0)),
        out_shape=jax.ShapeDtypeStruct((B, S, KPAD), jnp.int32),
    )(coordsT, centT)

    z, cnt, acc1 = pl.pallas_call(
        functools.partial(_c1_kernel, s_count=S, k=K, n=N),
        grid=(B,),
        in_specs=[
            pl.BlockSpec((1, N, 128), lambda b: (b, 0, 0)),
            pl.BlockSpec((1, S, KPAD), lambda b: (b, 0, 0)),
            pl.BlockSpec((64, 64), lambda b: (0, 0)),
            pl.BlockSpec((64, 64), lambda b: (0, 0)),
        ],
        out_specs=(
            pl.BlockSpec((1, N, 128), lambda b: (b, 0, 0)),
            pl.BlockSpec((1, 1, N), lambda b: (b, 0, 0)),
            pl.BlockSpec((1, 8, 128), lambda b: (0, 0, 0)),
        ),
        out_shape=(
            jax.ShapeDtypeStruct((B, N, 128), jnp.float32),
            jax.ShapeDtypeStruct((B, 1, N), jnp.float32),
            jax.ShapeDtypeStruct((1, 8, 128), jnp.float32),
        ),
    )(fk_feat, nidx, w1t, we1t)

    z2, acc2 = pl.pallas_call(
        functools.partial(_c2_kernel, total=total),
        grid=(B,),
        in_specs=[
            pl.BlockSpec((1, N, 128), lambda b: (b, 0, 0)),
            pl.BlockSpec((1, 1, N), lambda b: (b, 0, 0)),
            pl.BlockSpec((1, 8, 128), lambda b: (0, 0, 0)),
            pl.BlockSpec((1, 1, 128), lambda b: (0, 0, 0)),
            pl.BlockSpec((1, 1, 128), lambda b: (0, 0, 0)),
            pl.BlockSpec((64, 64), lambda b: (0, 0)),
            pl.BlockSpec((64, 64), lambda b: (0, 0)),
        ],
        out_specs=(
            pl.BlockSpec((1, N, 128), lambda b: (b, 0, 0)),
            pl.BlockSpec((1, 8, 128), lambda b: (0, 0, 0)),
        ),
        out_shape=(
            jax.ShapeDtypeStruct((B, N, 128), jnp.float32),
            jax.ShapeDtypeStruct((1, 8, 128), jnp.float32),
        ),
    )(z, cnt, acc1, g1, b1, w2t, we2t)

    fk = pl.pallas_call(
        functools.partial(_c3_kernel, s_count=S, k=K, n=N, total=total),
        grid=(B,),
        in_specs=[
            pl.BlockSpec((1, N, 128), lambda b: (b, 0, 0)),
            pl.BlockSpec((1, S, KPAD), lambda b: (b, 0, 0)),
            pl.BlockSpec((1, 8, 128), lambda b: (0, 0, 0)),
            pl.BlockSpec((1, 1, 128), lambda b: (0, 0, 0)),
            pl.BlockSpec((1, 1, 128), lambda b: (0, 0, 0)),
        ],
        out_specs=pl.BlockSpec((1, S, 128), lambda b: (b, 0, 0)),
        out_shape=jax.ShapeDtypeStruct((B, S, 128), jnp.float32),
    )(z2, nidx, acc2, g2, b2)

    fk1 = jnp.transpose(fk[:, :, 0:64], (0, 2, 1))
    fk2 = jnp.transpose(fk[:, :, 64:128], (0, 2, 1))
    return fk1, fk2, x[:, 128:192, :], x[:, 192:256, :]


def kernel(x, coords, conv1_w, bn1_g, bn1_b, conv2_w, bn2_g, bn2_b,
           convE1_w, bnE1_g, bnE1_b, convE2_w, bnE2_g, bnE2_b):
    return _run(x, coords, conv1_w, bn1_g, bn1_b, conv2_w, bn2_g, bn2_b,
                convE1_w, bnE1_g, bnE1_b, convE2_w, bnE2_g, bnE2_b)
